# baseline (speedup 1.0000x reference)
import functools

import jax
import jax.numpy as jnp
from jax.experimental import pallas as pl
from jax.experimental.pallas import tpu as pltpu

_TEMPERATURE = 0.1
_EPSILON = 0.05
_N_SK_ITERS = 3
_LOSS_SWAV_W = 1.0


def _pass1_body(nsteps, s0_ref, m_ref, alpha_ref, m_scr, p_scr):
    i = pl.program_id(0)

    @pl.when(i == 0)
    def _init():
        m_scr[...] = jnp.full(m_scr.shape, -jnp.inf, m_scr.dtype)
        p_scr[...] = jnp.zeros(p_scr.shape, p_scr.dtype)

    x = s0_ref[...] * (1.0 / _EPSILON)
    bm = jnp.max(x, axis=0, keepdims=True)
    m_old = m_scr[...]
    m_new = jnp.maximum(m_old, bm)
    p_scr[...] = p_scr[...] * jnp.exp(m_old - m_new) + jnp.sum(
        jnp.exp(x - m_new), axis=0, keepdims=True)
    m_scr[...] = m_new

    @pl.when(i == nsteps - 1)
    def _fin():
        m_ref[...] = m_scr[...]
        alpha_ref[...] = (1.0 / alpha_ref.shape[1]) / p_scr[...]


def _q_body(n_total, s0_ref, m_ref, a_ref, beta_ref):
    w = jnp.exp(s0_ref[...] * (1.0 / _EPSILON) - m_ref[...]) * a_ref[...]
    q = jnp.sum(w, axis=1, keepdims=True)
    beta_ref[...] = (1.0 / n_total) / q


def _p_body(nsteps, s0_ref, m_ref, beta_ref, alpha_ref, p_scr):
    i = pl.program_id(0)

    @pl.when(i == 0)
    def _init():
        p_scr[...] = jnp.zeros(p_scr.shape, p_scr.dtype)

    w = jnp.exp(s0_ref[...] * (1.0 / _EPSILON) - m_ref[...]) * beta_ref[...]
    p_scr[...] = p_scr[...] + jnp.sum(w, axis=0, keepdims=True)

    @pl.when(i == nsteps - 1)
    def _fin():
        alpha_ref[...] = (1.0 / alpha_ref.shape[1]) / p_scr[...]


def _loss_body(s0_ref, sv_ref, m_ref, a_ref, out_ref):
    i = pl.program_id(0)

    @pl.when(i == 0)
    def _init():
        out_ref[0, 0] = 0.0

    w = jnp.exp(s0_ref[...] * (1.0 / _EPSILON) - m_ref[...]) * a_ref[...]
    q = jnp.sum(w, axis=1)
    t = sv_ref[...] * (1.0 / _TEMPERATURE)
    num = jnp.sum(w * t, axis=1)
    mx = jnp.max(t, axis=1)
    lse = mx + jnp.log(jnp.sum(jnp.exp(t - mx[:, None]), axis=1))
    out_ref[0, 0] += jnp.sum(num / q - lse)


def _row_block(n):
    for d in range(min(n, 512), 7, -8):
        if n % d == 0:
            return d
    return n


def kernel(projs, scores, regions_idxs):
    del projs, regions_idxs
    view_m = scores.shape[0]
    k = scores.shape[-1]
    s0 = scores[0].reshape(-1, k)
    n = s0.shape[0]
    nb = _row_block(n)
    grid = n // nb

    s0_spec = pl.BlockSpec((nb, k), lambda i: (i, 0))
    vec_spec = pl.BlockSpec((1, k), lambda i: (0, 0))
    row_spec = pl.BlockSpec((nb, 1), lambda i: (i, 0))

    m, alpha = pl.pallas_call(
        functools.partial(_pass1_body, grid),
        grid=(grid,),
        in_specs=[s0_spec],
        out_specs=[vec_spec, vec_spec],
        out_shape=[jax.ShapeDtypeStruct((1, k), jnp.float32)] * 2,
        scratch_shapes=[pltpu.VMEM((1, k), jnp.float32)] * 2,
    )(s0)

    q_call = pl.pallas_call(
        functools.partial(_q_body, n),
        grid=(grid,),
        in_specs=[s0_spec, vec_spec, vec_spec],
        out_specs=row_spec,
        out_shape=jax.ShapeDtypeStruct((n, 1), jnp.float32),
    )
    p_call = pl.pallas_call(
        functools.partial(_p_body, grid),
        grid=(grid,),
        in_specs=[s0_spec, vec_spec, row_spec],
        out_specs=vec_spec,
        out_shape=jax.ShapeDtypeStruct((1, k), jnp.float32),
        scratch_shapes=[pltpu.VMEM((1, k), jnp.float32)],
    )
    for _ in range(_N_SK_ITERS - 1):
        beta = q_call(s0, m, alpha)
        alpha = p_call(s0, m, beta)

    loss = jnp.float32(0.0)
    for v in range(1, view_m):
        sv = scores[v].reshape(-1, k)
        out = pl.pallas_call(
            _loss_body,
            grid=(grid,),
            in_specs=[s0_spec, s0_spec, vec_spec, vec_spec],
            out_specs=pl.BlockSpec(memory_space=pltpu.SMEM),
            out_shape=jax.ShapeDtypeStruct((1, 1), jnp.float32),
        )(s0, sv, m, alpha)
        loss = loss - out[0, 0] / n
    loss = loss / (view_m - 1)
    return _LOSS_SWAV_W * loss


# speedup vs baseline: 1.1533x; 1.1533x over previous
import functools

import jax
import jax.numpy as jnp
from jax.experimental import pallas as pl
from jax.experimental.pallas import tpu as pltpu

_TEMPERATURE = 0.1
_EPSILON = 0.05
_N_SK_ITERS = 3
_LOSS_SWAV_W = 1.0


def _pass1_body(nsteps, s0_ref, a_ref, mb_ref, m_ref, alpha_ref, m_scr, p_scr):
    i = pl.program_id(0)

    @pl.when(i == 0)
    def _init():
        m_scr[...] = jnp.full(m_scr.shape, -jnp.inf, m_scr.dtype)
        p_scr[...] = jnp.zeros(p_scr.shape, p_scr.dtype)

    x = s0_ref[...] * (1.0 / _EPSILON)
    bm = jnp.max(x, axis=0, keepdims=True)
    m_old = m_scr[...]
    m_new = jnp.maximum(m_old, bm)
    e = jnp.exp(x - m_new)
    a_ref[...] = e.astype(a_ref.dtype)
    mb_ref[0] = m_new
    p_scr[...] = p_scr[...] * jnp.exp(m_old - m_new) + jnp.sum(
        e, axis=0, keepdims=True)
    m_scr[...] = m_new

    @pl.when(i == nsteps - 1)
    def _fin():
        m_ref[...] = m_scr[...]
        alpha_ref[...] = (1.0 / alpha_ref.shape[1]) / p_scr[...]


def _qp_body(nsteps, n_total, a_ref, mb_ref, m_ref, alpha_ref,
             alpha_out_ref, p_scr):
    i = pl.program_id(0)

    @pl.when(i == 0)
    def _init():
        p_scr[...] = jnp.zeros(p_scr.shape, p_scr.dtype)

    gamma = jnp.exp(mb_ref[0] - m_ref[...])
    a = a_ref[...].astype(jnp.float32)
    w = a * (gamma * alpha_ref[...])
    q = jnp.sum(w, axis=1, keepdims=True)
    beta = (1.0 / n_total) / q
    p_scr[...] = p_scr[...] + jnp.sum(a * beta, axis=0, keepdims=True) * gamma

    @pl.when(i == nsteps - 1)
    def _fin():
        alpha_out_ref[...] = (1.0 / alpha_out_ref.shape[1]) / p_scr[...]


def _loss_body(a_ref, sv_ref, mb_ref, m_ref, alpha_ref, out_ref):
    i = pl.program_id(0)

    @pl.when(i == 0)
    def _init():
        out_ref[0, 0] = 0.0

    gamma = jnp.exp(mb_ref[0] - m_ref[...])
    w = a_ref[...].astype(jnp.float32) * (gamma * alpha_ref[...])
    q = jnp.sum(w, axis=1)
    t = sv_ref[...] * (1.0 / _TEMPERATURE)
    num = jnp.sum(w * t, axis=1)
    mx = jnp.max(t, axis=1)
    lse = mx + jnp.log(jnp.sum(jnp.exp(t - mx[:, None]), axis=1))
    out_ref[0, 0] += jnp.sum(num / q - lse)


def _row_block(n):
    for d in range(min(n, 512), 7, -8):
        if n % d == 0:
            return d
    return n


def kernel(projs, scores, regions_idxs):
    del projs, regions_idxs
    view_m = scores.shape[0]
    k = scores.shape[-1]
    s0 = scores[0].reshape(-1, k)
    n = s0.shape[0]
    nb = _row_block(n)
    grid = n // nb

    blk_spec = pl.BlockSpec((nb, k), lambda i: (i, 0))
    vec_spec = pl.BlockSpec((1, k), lambda i: (0, 0))
    mb_spec = pl.BlockSpec((1, 1, k), lambda i: (i, 0, 0))

    a_mat, m_b, m, alpha = pl.pallas_call(
        functools.partial(_pass1_body, grid),
        grid=(grid,),
        in_specs=[blk_spec],
        out_specs=[blk_spec, mb_spec, vec_spec, vec_spec],
        out_shape=[
            jax.ShapeDtypeStruct((n, k), jnp.bfloat16),
            jax.ShapeDtypeStruct((grid, 1, k), jnp.float32),
            jax.ShapeDtypeStruct((1, k), jnp.float32),
            jax.ShapeDtypeStruct((1, k), jnp.float32),
        ],
        scratch_shapes=[pltpu.VMEM((1, k), jnp.float32)] * 2,
    )(s0)

    qp_call = pl.pallas_call(
        functools.partial(_qp_body, grid, n),
        grid=(grid,),
        in_specs=[blk_spec, mb_spec, vec_spec, vec_spec],
        out_specs=vec_spec,
        out_shape=jax.ShapeDtypeStruct((1, k), jnp.float32),
        scratch_shapes=[pltpu.VMEM((1, k), jnp.float32)],
    )
    for _ in range(_N_SK_ITERS - 1):
        alpha = qp_call(a_mat, m_b, m, alpha)

    loss = jnp.float32(0.0)
    for v in range(1, view_m):
        sv = scores[v].reshape(-1, k)
        out = pl.pallas_call(
            _loss_body,
            grid=(grid,),
            in_specs=[blk_spec, blk_spec, mb_spec, vec_spec, vec_spec],
            out_specs=pl.BlockSpec(memory_space=pltpu.SMEM),
            out_shape=jax.ShapeDtypeStruct((1, 1), jnp.float32),
        )(a_mat, sv, m_b, m, alpha)
        loss = loss - out[0, 0] / n
    loss = loss / (view_m - 1)
    return _LOSS_SWAV_W * loss


# speedup vs baseline: 1.1991x; 1.0397x over previous
import functools

import jax
import jax.numpy as jnp
from jax.experimental import pallas as pl
from jax.experimental.pallas import tpu as pltpu

_TEMPERATURE = 0.1
_EPSILON = 0.05
_N_SK_ITERS = 3
_LOSS_SWAV_W = 1.0
_PAD = -1e30


def _col_mask(k_real):
    c = jax.lax.broadcasted_iota(jnp.int32, (24, 128), 0)
    l = jax.lax.broadcasted_iota(jnp.int32, (24, 128), 1)
    return c * 128 + l < k_real


def _pass1_body(nsteps, k_real, s0_ref, a_ref, mb_ref, m_ref, alpha_ref,
                m_scr, p_scr):
    i = pl.program_id(0)

    @pl.when(i == 0)
    def _init():
        m_scr[...] = jnp.full(m_scr.shape, -jnp.inf, m_scr.dtype)
        p_scr[...] = jnp.zeros(p_scr.shape, p_scr.dtype)

    x = s0_ref[...] * (1.0 / _EPSILON)
    bm = jnp.max(x, axis=0)
    m_old = m_scr[...]
    m_new = jnp.maximum(m_old, bm)
    e = jnp.exp(x - m_new[None])
    nb = x.shape[0]
    a_ref[...] = e.astype(a_ref.dtype).reshape(nb // 2, 48, 128)
    mb_ref[0] = m_new
    p_scr[...] = p_scr[...] * jnp.exp(m_old - m_new) + jnp.sum(e, axis=0)
    m_scr[...] = m_new

    @pl.when(i == nsteps - 1)
    def _fin():
        m_ref[...] = m_scr[...]
        alpha_ref[...] = jnp.where(
            _col_mask(k_real), (1.0 / k_real) / p_scr[...], 0.0)


def _qp_body(nsteps, n_total, k_real, a_ref, mb_ref, m_ref, alpha_ref,
             alpha_out_ref, p_scr):
    i = pl.program_id(0)

    @pl.when(i == 0)
    def _init():
        p_scr[...] = jnp.zeros(p_scr.shape, p_scr.dtype)

    gamma = jnp.exp(mb_ref[0] - m_ref[...])
    nb2 = a_ref.shape[0]
    a = a_ref[...].astype(jnp.float32).reshape(nb2 * 2, 24, 128)
    w = a * (gamma * alpha_ref[...])[None]
    q = jnp.sum(w, axis=(1, 2))
    beta = (1.0 / n_total) / q
    p_scr[...] = p_scr[...] + jnp.sum(a * beta[:, None, None], axis=0) * gamma

    @pl.when(i == nsteps - 1)
    def _fin():
        alpha_out_ref[...] = jnp.where(
            _col_mask(k_real), (1.0 / k_real) / p_scr[...], 0.0)


def _loss_body(a_ref, sv_ref, mb_ref, m_ref, alpha_ref, out_ref):
    i = pl.program_id(0)

    @pl.when(i == 0)
    def _init():
        out_ref[0, 0] = 0.0

    gamma = jnp.exp(mb_ref[0] - m_ref[...])
    nb2 = a_ref.shape[0]
    a = a_ref[...].astype(jnp.float32).reshape(nb2 * 2, 24, 128)
    w = a * (gamma * alpha_ref[...])[None]
    q = jnp.sum(w, axis=(1, 2))
    t = sv_ref[...] * (1.0 / _TEMPERATURE)
    num = jnp.sum(w * t, axis=(1, 2))
    mx = jnp.max(t, axis=(1, 2))
    lse = mx + jnp.log(jnp.sum(jnp.exp(t - mx[:, None, None]), axis=(1, 2)))
    out_ref[0, 0] += jnp.sum(num / q - lse)


def kernel(projs, scores, regions_idxs):
    del projs, regions_idxs
    view_m = scores.shape[0]
    k = scores.shape[-1]
    n = scores.shape[1] * scores.shape[2]
    kp = ((k + 127) // 128) * 128
    xp = jnp.pad(
        scores, ((0, 0), (0, 0), (0, 0), (0, kp - k)),
        constant_values=jnp.float32(_PAD),
    ).reshape(view_m * n, kp // 128, 128)

    nb = 448
    grid = n // nb

    blk = pl.BlockSpec((nb, 24, 128), lambda i: (i, 0, 0))
    a_blk = pl.BlockSpec((nb // 2, 48, 128), lambda i: (i, 0, 0))
    vec = pl.BlockSpec((24, 128), lambda i: (0, 0))
    mb_spec = pl.BlockSpec((1, 24, 128), lambda i: (i, 0, 0))

    a_mat, m_b, m, alpha = pl.pallas_call(
        functools.partial(_pass1_body, grid, k),
        grid=(grid,),
        in_specs=[blk],
        out_specs=[a_blk, mb_spec, vec, vec],
        out_shape=[
            jax.ShapeDtypeStruct((n // 2, 48, 128), jnp.bfloat16),
            jax.ShapeDtypeStruct((grid, 24, 128), jnp.float32),
            jax.ShapeDtypeStruct((24, 128), jnp.float32),
            jax.ShapeDtypeStruct((24, 128), jnp.float32),
        ],
        scratch_shapes=[pltpu.VMEM((24, 128), jnp.float32)] * 2,
    )(xp)

    qp_call = pl.pallas_call(
        functools.partial(_qp_body, grid, n, k),
        grid=(grid,),
        in_specs=[a_blk, mb_spec, vec, vec],
        out_specs=vec,
        out_shape=jax.ShapeDtypeStruct((24, 128), jnp.float32),
        scratch_shapes=[pltpu.VMEM((24, 128), jnp.float32)],
    )
    for _ in range(_N_SK_ITERS - 1):
        alpha = qp_call(a_mat, m_b, m, alpha)

    loss = jnp.float32(0.0)
    for v in range(1, view_m):
        sv_blk = pl.BlockSpec((nb, 24, 128), lambda i, v=v: (i + v * grid, 0, 0))
        out = pl.pallas_call(
            _loss_body,
            grid=(grid,),
            in_specs=[a_blk, sv_blk, mb_spec, vec, vec],
            out_specs=pl.BlockSpec(memory_space=pltpu.SMEM),
            out_shape=jax.ShapeDtypeStruct((1, 1), jnp.float32),
        )(a_mat, xp, m_b, m, alpha)
        loss = loss - out[0, 0] / n
    loss = loss / (view_m - 1)
    return _LOSS_SWAV_W * loss


# speedup vs baseline: 1.2019x; 1.0023x over previous
import functools

import jax
import jax.numpy as jnp
from jax.experimental import pallas as pl
from jax.experimental.pallas import tpu as pltpu

_TEMPERATURE = 0.1
_EPSILON = 0.05
_N_SK_ITERS = 3
_LOSS_SWAV_W = 1.0
_PAD = -1e30


def _col_mask(k_real):
    c = jax.lax.broadcasted_iota(jnp.int32, (24, 128), 0)
    l = jax.lax.broadcasted_iota(jnp.int32, (24, 128), 1)
    return c * 128 + l < k_real


def _pass1_body(nsteps, k_real, s0_ref, a_ref, mb_ref, m_ref, alpha_ref,
                m_scr, p_scr):
    i = pl.program_id(0)

    @pl.when(i == 0)
    def _init():
        m_scr[...] = jnp.full(m_scr.shape, -jnp.inf, m_scr.dtype)
        p_scr[...] = jnp.zeros(p_scr.shape, p_scr.dtype)

    x = s0_ref[...] * (1.0 / _EPSILON)
    bm = jnp.max(x, axis=0)
    m_old = m_scr[...]
    m_new = jnp.maximum(m_old, bm)
    e = jnp.exp(x - m_new[None])
    nb = x.shape[0]
    a_ref[...] = e.astype(a_ref.dtype).reshape(nb // 2, 48, 128)
    mb_ref[0] = m_new
    p_scr[...] = p_scr[...] * jnp.exp(m_old - m_new) + jnp.sum(e, axis=0)
    m_scr[...] = m_new

    @pl.when(i == nsteps - 1)
    def _fin():
        m_ref[...] = m_scr[...]
        alpha_ref[...] = jnp.where(
            _col_mask(k_real), (1.0 / k_real) / p_scr[...], 0.0)


def _qp_body(nsteps, n_total, k_real, a_ref, mb_ref, m_ref, alpha_ref,
             alpha_out_ref, p_scr, al_scr):
    i = pl.program_id(0)
    j = i % nsteps

    @pl.when(i == 0)
    def _seed():
        al_scr[...] = alpha_ref[...]

    @pl.when(j == 0)
    def _init():
        p_scr[...] = jnp.zeros(p_scr.shape, p_scr.dtype)

    gamma = jnp.exp(mb_ref[0] - m_ref[...])
    nb2 = a_ref.shape[0]
    a = a_ref[...].astype(jnp.float32).reshape(nb2 * 2, 24, 128)
    w = a * (gamma * al_scr[...])[None]
    q = jnp.sum(w, axis=(1, 2))
    beta = (1.0 / n_total) / q
    p_scr[...] = p_scr[...] + jnp.sum(a * beta[:, None, None], axis=0) * gamma

    @pl.when(j == nsteps - 1)
    def _fin():
        nxt = jnp.where(_col_mask(k_real), (1.0 / k_real) / p_scr[...], 0.0)
        al_scr[...] = nxt
        alpha_out_ref[...] = nxt


def _loss_body(a_ref, sv_ref, mb_ref, m_ref, alpha_ref, out_ref):
    i = pl.program_id(0)

    @pl.when(i == 0)
    def _init():
        out_ref[0, 0] = 0.0

    gamma = jnp.exp(mb_ref[0] - m_ref[...])
    nb2 = a_ref.shape[0]
    a = a_ref[...].astype(jnp.float32).reshape(nb2 * 2, 24, 128)
    w = a * (gamma * alpha_ref[...])[None]
    q = jnp.sum(w, axis=(1, 2))
    t = sv_ref[...] * (1.0 / _TEMPERATURE)
    num = jnp.sum(w * t, axis=(1, 2))
    mx = jnp.max(t, axis=(1, 2))
    lse = mx + jnp.log(jnp.sum(jnp.exp(t - mx[:, None, None]), axis=(1, 2)))
    out_ref[0, 0] += jnp.sum(num / q - lse)


def kernel(projs, scores, regions_idxs):
    del projs, regions_idxs
    view_m = scores.shape[0]
    k = scores.shape[-1]
    n = scores.shape[1] * scores.shape[2]
    kp = ((k + 127) // 128) * 128
    xp = jnp.pad(
        scores, ((0, 0), (0, 0), (0, 0), (0, kp - k)),
        constant_values=jnp.float32(_PAD),
    ).reshape(view_m * n, kp // 128, 128)

    nb = 448
    grid = n // nb

    blk = pl.BlockSpec((nb, 24, 128), lambda i: (i, 0, 0))
    a_blk = pl.BlockSpec((nb // 2, 48, 128), lambda i: (i, 0, 0))
    vec = pl.BlockSpec((24, 128), lambda i: (0, 0))
    mb_spec = pl.BlockSpec((1, 24, 128), lambda i: (i, 0, 0))

    a_mat, m_b, m, alpha = pl.pallas_call(
        functools.partial(_pass1_body, grid, k),
        grid=(grid,),
        in_specs=[blk],
        out_specs=[a_blk, mb_spec, vec, vec],
        out_shape=[
            jax.ShapeDtypeStruct((n // 2, 48, 128), jnp.bfloat16),
            jax.ShapeDtypeStruct((grid, 24, 128), jnp.float32),
            jax.ShapeDtypeStruct((24, 128), jnp.float32),
            jax.ShapeDtypeStruct((24, 128), jnp.float32),
        ],
        scratch_shapes=[pltpu.VMEM((24, 128), jnp.float32)] * 2,
    )(xp)

    n_it = _N_SK_ITERS - 1
    qp_call = pl.pallas_call(
        functools.partial(_qp_body, grid, n, k),
        grid=(n_it * grid,),
        in_specs=[
            pl.BlockSpec((nb // 2, 48, 128), lambda i: (i % 14, 0, 0)),
            pl.BlockSpec((1, 24, 128), lambda i: (i % 14, 0, 0)),
            vec, vec],
        out_specs=vec,
        out_shape=jax.ShapeDtypeStruct((24, 128), jnp.float32),
        scratch_shapes=[pltpu.VMEM((24, 128), jnp.float32)] * 2,
    )
    alpha = qp_call(a_mat, m_b, m, alpha)

    loss = jnp.float32(0.0)
    for v in range(1, view_m):
        sv_blk = pl.BlockSpec((nb, 24, 128), lambda i, v=v: (i + v * grid, 0, 0))
        out = pl.pallas_call(
            _loss_body,
            grid=(grid,),
            in_specs=[a_blk, sv_blk, mb_spec, vec, vec],
            out_specs=pl.BlockSpec(memory_space=pltpu.SMEM),
            out_shape=jax.ShapeDtypeStruct((1, 1), jnp.float32),
        )(a_mat, xp, m_b, m, alpha)
        loss = loss - out[0, 0] / n
    loss = loss / (view_m - 1)
    return _LOSS_SWAV_W * loss


# speedup vs baseline: 1.2601x; 1.0484x over previous
import functools

import jax
import jax.numpy as jnp
from jax.experimental import pallas as pl
from jax.experimental.pallas import tpu as pltpu

_TEMPERATURE = 0.1
_EPSILON = 0.05
_N_SK_ITERS = 3
_LOSS_SWAV_W = 1.0
_PAD = -1e30


def _col_mask(k_real):
    c = jax.lax.broadcasted_iota(jnp.int32, (24, 128), 0)
    l = jax.lax.broadcasted_iota(jnp.int32, (24, 128), 1)
    return c * 128 + l < k_real


def _pass1_body(nsteps, k_real, s0_ref, a_ref, mb_ref, m_ref, alpha_ref,
                m_scr, p_scr):
    i = pl.program_id(0)

    @pl.when(i == 0)
    def _init():
        m_scr[...] = jnp.full(m_scr.shape, -jnp.inf, m_scr.dtype)
        p_scr[...] = jnp.zeros(p_scr.shape, p_scr.dtype)

    x = s0_ref[...] * (1.0 / _EPSILON)
    bm = jnp.max(x, axis=0)
    m_old = m_scr[...]
    m_new = jnp.maximum(m_old, bm)
    e = jnp.exp(x - m_new[None])
    nb = x.shape[0]
    a_ref[...] = e.astype(a_ref.dtype).reshape(nb // 2, 48, 128)
    mb_ref[0] = m_new
    p_scr[...] = p_scr[...] * jnp.exp(m_old - m_new) + jnp.sum(e, axis=0)
    m_scr[...] = m_new

    @pl.when(i == nsteps - 1)
    def _fin():
        m_ref[...] = m_scr[...]
        alpha_ref[...] = jnp.where(
            _col_mask(k_real), (1.0 / k_real) / p_scr[...], 0.0)


def _qp_body(nsteps, n_total, k_real, a_ref, mb_ref, m_ref, alpha_ref,
             alpha_out_ref, p_scr, al_scr):
    i = pl.program_id(0)
    j = i % nsteps

    @pl.when(i == 0)
    def _seed():
        al_scr[...] = alpha_ref[...]

    @pl.when(j == 0)
    def _init():
        p_scr[...] = jnp.zeros(p_scr.shape, p_scr.dtype)

    gamma = jnp.exp(mb_ref[0] - m_ref[...])
    nb2 = a_ref.shape[0]
    a = a_ref[...].astype(jnp.float32).reshape(nb2 * 2, 24, 128)
    w = a * (gamma * al_scr[...])[None]
    q = jnp.sum(w, axis=(1, 2))
    beta = (1.0 / n_total) / q
    p_scr[...] = p_scr[...] + jnp.sum(a * beta[:, None, None], axis=0) * gamma

    @pl.when(j == nsteps - 1)
    def _fin():
        nxt = jnp.where(_col_mask(k_real), (1.0 / k_real) / p_scr[...], 0.0)
        al_scr[...] = nxt
        alpha_out_ref[...] = nxt


def _loss_body(a_ref, sv_ref, mb_ref, m_ref, alpha_ref, out_ref):
    i = pl.program_id(0)

    @pl.when(i == 0)
    def _init():
        out_ref[0, 0] = 0.0

    gamma = jnp.exp(mb_ref[0] - m_ref[...])
    nb2 = a_ref.shape[0]
    a = a_ref[...].astype(jnp.float32).reshape(nb2 * 2, 24, 128)
    w = a * (gamma * alpha_ref[...])[None]
    q = jnp.sum(w, axis=(1, 2))
    t = sv_ref[...] * (1.0 / _TEMPERATURE)
    num = jnp.sum(w * t, axis=(1, 2))
    mx = jnp.max(t, axis=(1, 2))
    lse = mx + jnp.log(jnp.sum(jnp.exp(t - mx[:, None, None]), axis=(1, 2)))
    out_ref[0, 0] += jnp.sum(num / q - lse)


def kernel(projs, scores, regions_idxs):
    del projs, regions_idxs
    view_m = scores.shape[0]
    k = scores.shape[-1]
    n = scores.shape[1] * scores.shape[2]
    kp = ((k + 127) // 128) * 128
    xp5 = jnp.pad(
        scores, ((0, 0), (0, 0), (0, 0), (0, kp - k)),
        constant_values=jnp.float32(_PAD),
    ).reshape(view_m, scores.shape[1], scores.shape[2], kp // 128, 128)
    xp5 = jax.lax.optimization_barrier(xp5)
    xp = xp5.reshape(view_m * n, kp // 128, 128)

    nb = 448
    grid = n // nb

    blk = pl.BlockSpec((nb, 24, 128), lambda i: (i, 0, 0))
    a_blk = pl.BlockSpec((nb // 2, 48, 128), lambda i: (i, 0, 0))
    vec = pl.BlockSpec((24, 128), lambda i: (0, 0))
    mb_spec = pl.BlockSpec((1, 24, 128), lambda i: (i, 0, 0))

    a_mat, m_b, m, alpha = pl.pallas_call(
        functools.partial(_pass1_body, grid, k),
        grid=(grid,),
        in_specs=[blk],
        out_specs=[a_blk, mb_spec, vec, vec],
        out_shape=[
            jax.ShapeDtypeStruct((n // 2, 48, 128), jnp.bfloat16),
            jax.ShapeDtypeStruct((grid, 24, 128), jnp.float32),
            jax.ShapeDtypeStruct((24, 128), jnp.float32),
            jax.ShapeDtypeStruct((24, 128), jnp.float32),
        ],
        scratch_shapes=[pltpu.VMEM((24, 128), jnp.float32)] * 2,
    )(xp)

    n_it = _N_SK_ITERS - 1
    qp_call = pl.pallas_call(
        functools.partial(_qp_body, grid, n, k),
        grid=(n_it * grid,),
        in_specs=[
            pl.BlockSpec((nb // 2, 48, 128), lambda i: (i % 14, 0, 0)),
            pl.BlockSpec((1, 24, 128), lambda i: (i % 14, 0, 0)),
            vec, vec],
        out_specs=vec,
        out_shape=jax.ShapeDtypeStruct((24, 128), jnp.float32),
        scratch_shapes=[pltpu.VMEM((24, 128), jnp.float32)] * 2,
    )
    alpha = qp_call(a_mat, m_b, m, alpha)

    loss = jnp.float32(0.0)
    for v in range(1, view_m):
        sv_blk = pl.BlockSpec((nb, 24, 128), lambda i, v=v: (i + v * grid, 0, 0))
        out = pl.pallas_call(
            _loss_body,
            grid=(grid,),
            in_specs=[a_blk, sv_blk, mb_spec, vec, vec],
            out_specs=pl.BlockSpec(memory_space=pltpu.SMEM),
            out_shape=jax.ShapeDtypeStruct((1, 1), jnp.float32),
        )(a_mat, xp, m_b, m, alpha)
        loss = loss - out[0, 0] / n
    loss = loss / (view_m - 1)
    return _LOSS_SWAV_W * loss
